# baseline (speedup 1.0000x reference)
import functools

import jax
import jax.numpy as jnp
from jax import lax
from jax.experimental import pallas as pl
from jax.experimental.pallas import tpu as pltpu
from jax.experimental.pallas import tpu_sc as plsc

_N = 10000
_E = 320000
_D = 128
_H = 16
_C = 7

_NC = 2
_NS = 16
_NW = _NC * _NS
_CB = 128
_KC = 80
_EPT = _KC * _CB
_EPAD = _NW * _EPT
_NTRASH = _N
_RPT = _N // _NS

_SLAG = 4
_ACCN = _N + 8

_mesh = plsc.VectorSubcoreMesh(core_axis_name="c", subcore_axis_name="s")


def _zero_rows(buf, nrows):
    def body(i, _):
        buf[i] = jnp.zeros((16,), jnp.float32)
        return 0
    lax.fori_loop(0, nrows, body, 0)


def _acc_prologue(sid, dump_v, acc_sh):
    _zero_rows(dump_v, _RPT)
    pltpu.sync_copy(dump_v, acc_sh.at[pl.ds(sid * _RPT, _RPT)])
    plsc.subcore_barrier()


def _acc_epilogue(cid, sid, dump_v, acc_sh, out):
    plsc.subcore_barrier()
    pltpu.sync_copy(acc_sh.at[pl.ds(sid * _RPT, _RPT)], dump_v)
    pltpu.sync_copy(dump_v, out.at[cid, sid])


def _sc_deg_body(dst_h, out, dst_v, rows_v, dump_v, acc_sh, ssem):
    cid = lax.axis_index("c")
    sid = lax.axis_index("s")
    wid = sid * _NC + cid
    _acc_prologue(sid, dump_v, acc_sh)
    pltpu.sync_copy(dst_h.at[wid], dst_v)

    def fill(i, _):
        rows_v[i] = jnp.ones((16,), jnp.float32)
        return 0
    lax.fori_loop(0, _CB, fill, 0)

    def body(j, _):
        pltpu.async_copy(rows_v, acc_sh.at[dst_v.at[j]], ssem, add=True)

        @pl.when(j >= _SLAG)
        def _wait_old():
            pltpu.make_async_copy(out.at[0, 0, pl.ds(0, _CB)], rows_v,
                                  ssem).wait()
        return 0
    lax.fori_loop(0, _KC, body, 0)

    def drain(j, _):
        pltpu.make_async_copy(out.at[0, 0, pl.ds(0, _CB)], rows_v,
                              ssem).wait()
        return 0
    lax.fori_loop(0, _SLAG, drain, 0)
    _acc_epilogue(cid, sid, dump_v, acc_sh, out)


_NB = 4


def _sc_gs_body(table, src_h, dst_h, out, src_v, dst_v, rows_v, dump_v,
                acc_sh, gsem, ssem):
    cid = lax.axis_index("c")
    sid = lax.axis_index("s")
    wid = sid * _NC + cid
    _acc_prologue(sid, dump_v, acc_sh)
    pltpu.sync_copy(src_h.at[wid], src_v)
    pltpu.sync_copy(dst_h.at[wid], dst_v)

    for b in range(_NB - 1):
        pltpu.async_copy(table.at[src_v.at[b]], rows_v.at[b], gsem)

    def body(j, _):
        b = lax.rem(j, _NB)
        pltpu.make_async_copy(table.at[pl.ds(0, _CB)], rows_v.at[b],
                              gsem).wait()
        pltpu.async_copy(rows_v.at[b], acc_sh.at[dst_v.at[j]], ssem,
                         add=True)

        @pl.when(j >= 1)
        def _wait_prev_scatter():
            pltpu.make_async_copy(table.at[pl.ds(0, _CB)], rows_v.at[0],
                                  ssem).wait()

        nj = j + _NB - 1

        @pl.when(nj < _KC)
        def _prefetch():
            pltpu.async_copy(table.at[src_v.at[nj]],
                             rows_v.at[lax.rem(nj, _NB)], gsem)
        return 0
    lax.fori_loop(0, _KC, body, 0)
    pltpu.make_async_copy(table.at[pl.ds(0, _CB)], rows_v.at[0], ssem).wait()
    _acc_epilogue(cid, sid, dump_v, acc_sh, out)


_sc_deg = pl.kernel(
    _sc_deg_body,
    out_type=jax.ShapeDtypeStruct((_NC, _NS, _RPT, 16), jnp.float32),
    mesh=_mesh,
    compiler_params=pltpu.CompilerParams(use_tc_tiling_on_sc=False),
    scratch_types=[
        pltpu.VMEM((_KC, _CB), jnp.int32),
        pltpu.VMEM((_CB, 16), jnp.float32),
        pltpu.VMEM((_RPT, 16), jnp.float32),
        pltpu.VMEM_SHARED((_ACCN, 16), jnp.float32),
        pltpu.SemaphoreType.DMA,
    ],
)

_sc_gs = pl.kernel(
    _sc_gs_body,
    out_type=jax.ShapeDtypeStruct((_NC, _NS, _RPT, 16), jnp.float32),
    mesh=_mesh,
    compiler_params=pltpu.CompilerParams(use_tc_tiling_on_sc=False),
    scratch_types=[
        pltpu.VMEM((_KC, _CB), jnp.int32),
        pltpu.VMEM((_KC, _CB), jnp.int32),
        pltpu.VMEM((_NB, _CB, 16), jnp.float32),
        pltpu.VMEM((_RPT, 16), jnp.float32),
        pltpu.VMEM_SHARED((_ACCN, 16), jnp.float32),
        pltpu.SemaphoreType.DMA,
        pltpu.SemaphoreType.DMA,
    ],
)


_RB = 1000
_GRID = _N // _RB


def _tc_prep_body(x_ref, w1_ref, degp_ref, dinv_ref, xws_ref):
    deg = degp_ref[0, :, 0:1] + degp_ref[1, :, 0:1] + 1.0
    dinv = lax.rsqrt(deg)
    xw = jnp.dot(x_ref[...], w1_ref[...], preferred_element_type=jnp.float32)
    dinv_ref[...] = dinv
    xws_ref[...] = xw * dinv


_tc_prep = pl.pallas_call(
    _tc_prep_body,
    grid=(_GRID,),
    in_specs=[
        pl.BlockSpec((_RB, _D), lambda i: (i, 0)),
        pl.BlockSpec((_D, _H), lambda i: (0, 0)),
        pl.BlockSpec((_NC, _RB, 16), lambda i: (0, i, 0)),
    ],
    out_specs=[
        pl.BlockSpec((_RB, 1), lambda i: (i, 0)),
        pl.BlockSpec((_RB, _H), lambda i: (i, 0)),
    ],
    out_shape=[
        jax.ShapeDtypeStruct((_N, 1), jnp.float32),
        jax.ShapeDtypeStruct((_N, _H), jnp.float32),
    ],
)


def _tc_layer_body(p_ref, dinv_ref, xws_ref, b1_ref, w2_ref, h_ref, hws_ref):
    dinv = dinv_ref[...]
    s = (p_ref[0] + p_ref[1] + xws_ref[...]) * dinv
    h = jnp.maximum(s + b1_ref[...], 0.0)
    h_ref[...] = h
    hws_ref[...] = jnp.dot(h, w2_ref[...],
                           preferred_element_type=jnp.float32) * dinv


_tc_layer = pl.pallas_call(
    _tc_layer_body,
    grid=(_GRID,),
    in_specs=[
        pl.BlockSpec((_NC, _RB, 16), lambda i: (0, i, 0)),
        pl.BlockSpec((_RB, 1), lambda i: (i, 0)),
        pl.BlockSpec((_RB, _H), lambda i: (i, 0)),
        pl.BlockSpec((1, _H), lambda i: (0, 0)),
        pl.BlockSpec((_H, 16), lambda i: (0, 0)),
    ],
    out_specs=[
        pl.BlockSpec((_RB, _H), lambda i: (i, 0)),
        pl.BlockSpec((_RB, 16), lambda i: (i, 0)),
    ],
    out_shape=[
        jax.ShapeDtypeStruct((_N, _H), jnp.float32),
        jax.ShapeDtypeStruct((_N, 16), jnp.float32),
    ],
)


def _tc_final_body(q_ref, dinv_ref, hws_ref, b2_ref, z_ref):
    z_ref[...] = ((q_ref[0] + q_ref[1] + hws_ref[...]) * dinv_ref[...]
                  + b2_ref[...])


_tc_final = pl.pallas_call(
    _tc_final_body,
    grid=(_GRID,),
    in_specs=[
        pl.BlockSpec((_NC, _RB, 16), lambda i: (0, i, 0)),
        pl.BlockSpec((_RB, 1), lambda i: (i, 0)),
        pl.BlockSpec((_RB, 16), lambda i: (i, 0)),
        pl.BlockSpec((1, 16), lambda i: (0, 0)),
    ],
    out_specs=pl.BlockSpec((_RB, 16), lambda i: (i, 0)),
    out_shape=jax.ShapeDtypeStruct((_N, 16), jnp.float32),
)


def kernel(x, edge_index, W1, b1, W2, b2):
    ei = edge_index.astype(jnp.int32)
    npad = _EPAD - _E
    src3 = jnp.concatenate(
        [ei[0], jnp.zeros((npad,), jnp.int32)]).reshape(_NW, _KC, _CB)
    dst3 = jnp.concatenate(
        [ei[1], jnp.full((npad,), _NTRASH, jnp.int32)]).reshape(_NW, _KC, _CB)

    b1r = b1.reshape(1, _H)
    W2p = jnp.zeros((_H, 16), jnp.float32).at[:, :_C].set(W2)
    b2p = jnp.zeros((1, 16), jnp.float32).at[0, :_C].set(b2)

    degp = _sc_deg(dst3).reshape(_NC, _N, 16)
    dinv, xws = _tc_prep(x, W1, degp)
    p = _sc_gs(xws, src3, dst3).reshape(_NC, _N, 16)
    h, hws = _tc_layer(p, dinv, xws, b1r, W2p)
    q = _sc_gs(hws, src3, dst3).reshape(_NC, _N, 16)
    z = _tc_final(q, dinv, hws, b2p)[:, :_C]
    return (h, z)


# speedup vs baseline: 1.0237x; 1.0237x over previous
import functools

import jax
import jax.numpy as jnp
from jax import lax
from jax.experimental import pallas as pl
from jax.experimental.pallas import tpu as pltpu
from jax.experimental.pallas import tpu_sc as plsc

_N = 10000
_E = 320000
_D = 128
_H = 16
_C = 7

_NC = 2
_NS = 16
_NW = _NC * _NS
_CB = 128
_KC = 80
_EPT = _KC * _CB
_EPAD = _NW * _EPT
_NTRASH = 512
_RPT = _N // _NS

_SLAG = 4
_ACCN = _N + _NTRASH

_mesh = plsc.VectorSubcoreMesh(core_axis_name="c", subcore_axis_name="s")


def _zero_rows(buf, nrows):
    def body(i, _):
        buf[i] = jnp.zeros((16,), jnp.float32)
        return 0
    lax.fori_loop(0, nrows, body, 0)


def _acc_prologue(sid, dump_v, acc_sh):
    _zero_rows(dump_v, _RPT)
    pltpu.sync_copy(dump_v, acc_sh.at[pl.ds(sid * _RPT, _RPT)])
    plsc.subcore_barrier()


def _acc_epilogue(cid, sid, dump_v, acc_sh, out):
    plsc.subcore_barrier()
    pltpu.sync_copy(acc_sh.at[pl.ds(sid * _RPT, _RPT)], dump_v)
    pltpu.sync_copy(dump_v, out.at[cid, sid])


def _sc_deg_body(dst_h, out, dst_v, rows_v, dump_v, acc_sh, ssem):
    cid = lax.axis_index("c")
    sid = lax.axis_index("s")
    wid = sid * _NC + cid
    _acc_prologue(sid, dump_v, acc_sh)
    pltpu.sync_copy(dst_h.at[wid], dst_v)

    def fill(i, _):
        rows_v[i] = jnp.ones((16,), jnp.float32)
        return 0
    lax.fori_loop(0, _CB, fill, 0)

    def body(j, _):
        pltpu.async_copy(rows_v, acc_sh.at[dst_v.at[j]], ssem, add=True)

        @pl.when(j >= _SLAG)
        def _wait_old():
            pltpu.make_async_copy(out.at[0, 0, pl.ds(0, _CB)], rows_v,
                                  ssem).wait()
        return 0
    lax.fori_loop(0, _KC, body, 0)

    def drain(j, _):
        pltpu.make_async_copy(out.at[0, 0, pl.ds(0, _CB)], rows_v,
                              ssem).wait()
        return 0
    lax.fori_loop(0, _SLAG, drain, 0)
    _acc_epilogue(cid, sid, dump_v, acc_sh, out)


_NB = 4


def _sc_gs_body(table, src_h, dst_h, out, src_v, dst_v, rows_v, dump_v,
                acc_sh, gsem, ssem):
    cid = lax.axis_index("c")
    sid = lax.axis_index("s")
    wid = sid * _NC + cid
    _acc_prologue(sid, dump_v, acc_sh)
    pltpu.sync_copy(src_h.at[wid], src_v)
    pltpu.sync_copy(dst_h.at[wid], dst_v)

    for b in range(_NB - 1):
        pltpu.async_copy(table.at[src_v.at[b]], rows_v.at[b], gsem)

    def body(j, _):
        b = lax.rem(j, _NB)
        pltpu.make_async_copy(table.at[pl.ds(0, _CB)], rows_v.at[b],
                              gsem).wait()
        pltpu.async_copy(rows_v.at[b], acc_sh.at[dst_v.at[j]], ssem,
                         add=True)

        @pl.when(j >= 1)
        def _wait_prev_scatter():
            pltpu.make_async_copy(table.at[pl.ds(0, _CB)], rows_v.at[0],
                                  ssem).wait()

        nj = j + _NB - 1

        @pl.when(nj < _KC)
        def _prefetch():
            pltpu.async_copy(table.at[src_v.at[nj]],
                             rows_v.at[lax.rem(nj, _NB)], gsem)
        return 0
    lax.fori_loop(0, _KC, body, 0)
    pltpu.make_async_copy(table.at[pl.ds(0, _CB)], rows_v.at[0], ssem).wait()
    _acc_epilogue(cid, sid, dump_v, acc_sh, out)


_sc_deg = pl.kernel(
    _sc_deg_body,
    out_type=jax.ShapeDtypeStruct((_NC, _NS, _RPT, 16), jnp.float32),
    mesh=_mesh,
    compiler_params=pltpu.CompilerParams(use_tc_tiling_on_sc=False),
    scratch_types=[
        pltpu.VMEM((_KC, _CB), jnp.int32),
        pltpu.VMEM((_CB, 16), jnp.float32),
        pltpu.VMEM((_RPT, 16), jnp.float32),
        pltpu.VMEM_SHARED((_ACCN, 16), jnp.float32),
        pltpu.SemaphoreType.DMA,
    ],
)

_sc_gs = pl.kernel(
    _sc_gs_body,
    out_type=jax.ShapeDtypeStruct((_NC, _NS, _RPT, 16), jnp.float32),
    mesh=_mesh,
    compiler_params=pltpu.CompilerParams(use_tc_tiling_on_sc=False),
    scratch_types=[
        pltpu.VMEM((_KC, _CB), jnp.int32),
        pltpu.VMEM((_KC, _CB), jnp.int32),
        pltpu.VMEM((_NB, _CB, 16), jnp.float32),
        pltpu.VMEM((_RPT, 16), jnp.float32),
        pltpu.VMEM_SHARED((_ACCN, 16), jnp.float32),
        pltpu.SemaphoreType.DMA,
        pltpu.SemaphoreType.DMA,
    ],
)


_RB = 1000
_GRID = _N // _RB


def _tc_prep_body(x_ref, w1_ref, degp_ref, dinv_ref, xws_ref):
    deg = degp_ref[0, :, 0:1] + degp_ref[1, :, 0:1] + 1.0
    dinv = lax.rsqrt(deg)
    xw = jnp.dot(x_ref[...], w1_ref[...], preferred_element_type=jnp.float32)
    dinv_ref[...] = dinv
    xws_ref[...] = xw * dinv


_tc_prep = pl.pallas_call(
    _tc_prep_body,
    grid=(_GRID,),
    in_specs=[
        pl.BlockSpec((_RB, _D), lambda i: (i, 0)),
        pl.BlockSpec((_D, _H), lambda i: (0, 0)),
        pl.BlockSpec((_NC, _RB, 16), lambda i: (0, i, 0)),
    ],
    out_specs=[
        pl.BlockSpec((_RB, 1), lambda i: (i, 0)),
        pl.BlockSpec((_RB, _H), lambda i: (i, 0)),
    ],
    out_shape=[
        jax.ShapeDtypeStruct((_N, 1), jnp.float32),
        jax.ShapeDtypeStruct((_N, _H), jnp.float32),
    ],
)


def _tc_layer_body(p_ref, dinv_ref, xws_ref, b1_ref, w2_ref, h_ref, hws_ref):
    dinv = dinv_ref[...]
    s = (p_ref[0] + p_ref[1] + xws_ref[...]) * dinv
    h = jnp.maximum(s + b1_ref[...], 0.0)
    h_ref[...] = h
    hws_ref[...] = jnp.dot(h, w2_ref[...],
                           preferred_element_type=jnp.float32) * dinv


_tc_layer = pl.pallas_call(
    _tc_layer_body,
    grid=(_GRID,),
    in_specs=[
        pl.BlockSpec((_NC, _RB, 16), lambda i: (0, i, 0)),
        pl.BlockSpec((_RB, 1), lambda i: (i, 0)),
        pl.BlockSpec((_RB, _H), lambda i: (i, 0)),
        pl.BlockSpec((1, _H), lambda i: (0, 0)),
        pl.BlockSpec((_H, 16), lambda i: (0, 0)),
    ],
    out_specs=[
        pl.BlockSpec((_RB, _H), lambda i: (i, 0)),
        pl.BlockSpec((_RB, 16), lambda i: (i, 0)),
    ],
    out_shape=[
        jax.ShapeDtypeStruct((_N, _H), jnp.float32),
        jax.ShapeDtypeStruct((_N, 16), jnp.float32),
    ],
)


def _tc_final_body(q_ref, dinv_ref, hws_ref, b2_ref, z_ref):
    z_ref[...] = ((q_ref[0] + q_ref[1] + hws_ref[...]) * dinv_ref[...]
                  + b2_ref[...])


_tc_final = pl.pallas_call(
    _tc_final_body,
    grid=(_GRID,),
    in_specs=[
        pl.BlockSpec((_NC, _RB, 16), lambda i: (0, i, 0)),
        pl.BlockSpec((_RB, 1), lambda i: (i, 0)),
        pl.BlockSpec((_RB, 16), lambda i: (i, 0)),
        pl.BlockSpec((1, 16), lambda i: (0, 0)),
    ],
    out_specs=pl.BlockSpec((_RB, 16), lambda i: (i, 0)),
    out_shape=jax.ShapeDtypeStruct((_N, 16), jnp.float32),
)


def kernel(x, edge_index, W1, b1, W2, b2):
    ei = edge_index.astype(jnp.int32)
    npad = _EPAD - _E
    src3 = jnp.concatenate(
        [ei[0], jnp.zeros((npad,), jnp.int32)]).reshape(_NW, _KC, _CB)
    pad_dst = _N + (jnp.arange(npad, dtype=jnp.int32) % _NTRASH)
    dst3 = jnp.concatenate([ei[1], pad_dst]).reshape(_NW, _KC, _CB)

    b1r = b1.reshape(1, _H)
    W2p = jnp.zeros((_H, 16), jnp.float32).at[:, :_C].set(W2)
    b2p = jnp.zeros((1, 16), jnp.float32).at[0, :_C].set(b2)

    degp = _sc_deg(dst3).reshape(_NC, _N, 16)
    dinv, xws = _tc_prep(x, W1, degp)
    p = _sc_gs(xws, src3, dst3).reshape(_NC, _N, 16)
    h, hws = _tc_layer(p, dinv, xws, b1r, W2p)
    q = _sc_gs(hws, src3, dst3).reshape(_NC, _N, 16)
    z = _tc_final(q, dinv, hws, b2p)[:, :_C]
    return (h, z)


# speedup vs baseline: 1.2994x; 1.2693x over previous
import functools

import jax
import jax.numpy as jnp
from jax import lax
from jax.experimental import pallas as pl
from jax.experimental.pallas import tpu as pltpu
from jax.experimental.pallas import tpu_sc as plsc

_N = 10000
_E = 320000
_D = 128
_H = 16
_C = 7

_NC = 2
_NS = 16
_NW = _NC * _NS
_CB = 128
_KC = 80
_EPT = _KC * _CB
_EPAD = _NW * _EPT
_NTRASH = 512
_RPT = _N // _NS

_SLAG = 4
_ACCN = _N + _NTRASH

_mesh = plsc.VectorSubcoreMesh(core_axis_name="c", subcore_axis_name="s")


def _zero_rows(buf, nrows):
    def body(i, _):
        buf[i] = jnp.zeros((16,), jnp.float32)
        return 0
    lax.fori_loop(0, nrows, body, 0)


def _acc_prologue(sid, dump_v, acc_sh):
    _zero_rows(dump_v, _RPT)
    pltpu.sync_copy(dump_v, acc_sh.at[pl.ds(sid * _RPT, _RPT)])
    plsc.subcore_barrier()


def _acc_epilogue(cid, sid, dump_v, acc_sh, out):
    plsc.subcore_barrier()
    pltpu.sync_copy(acc_sh.at[pl.ds(sid * _RPT, _RPT)], dump_v)
    pltpu.sync_copy(dump_v, out.at[cid, sid])


def _sc_deg_body(dst_h, out, dst_v, rows_v, dump_v, acc_sh, ssem):
    cid = lax.axis_index("c")
    sid = lax.axis_index("s")
    wid = sid * _NC + cid
    _acc_prologue(sid, dump_v, acc_sh)
    pltpu.sync_copy(dst_h.at[wid], dst_v)

    def fill(i, _):
        rows_v[i] = jnp.ones((16,), jnp.float32)
        return 0
    lax.fori_loop(0, _CB, fill, 0)

    def body(j, _):
        pltpu.async_copy(rows_v, acc_sh.at[dst_v.at[j]], ssem, add=True)

        @pl.when(j >= _SLAG)
        def _wait_old():
            pltpu.make_async_copy(out.at[0, 0, pl.ds(0, _CB)], rows_v,
                                  ssem).wait()
        return 0
    lax.fori_loop(0, _KC, body, 0)

    def drain(j, _):
        pltpu.make_async_copy(out.at[0, 0, pl.ds(0, _CB)], rows_v,
                              ssem).wait()
        return 0
    lax.fori_loop(0, _SLAG, drain, 0)
    _acc_epilogue(cid, sid, dump_v, acc_sh, out)


_NB = 4


def _sc_gs_body(table, src_h, dst_h, out, src_v, dst_v, rows_v, dump_v,
                acc_sh, gsem, ssem):
    cid = lax.axis_index("c")
    sid = lax.axis_index("s")
    wid = sid * _NC + cid
    _acc_prologue(sid, dump_v, acc_sh)
    pltpu.sync_copy(src_h.at[wid], src_v)
    pltpu.sync_copy(dst_h.at[wid], dst_v)

    for b in range(_NB - 1):
        pltpu.async_copy(table.at[src_v.at[b]], rows_v.at[b], gsem)

    def body(j, _):
        b = lax.rem(j, _NB)
        pltpu.make_async_copy(table.at[pl.ds(0, _CB)], rows_v.at[b],
                              gsem).wait()
        pltpu.async_copy(rows_v.at[b], acc_sh.at[dst_v.at[j]], ssem,
                         add=True)

        @pl.when(j >= 1)
        def _wait_prev_scatter():
            pltpu.make_async_copy(table.at[pl.ds(0, _CB)], rows_v.at[0],
                                  ssem).wait()

        nj = j + _NB - 1

        @pl.when(nj < _KC)
        def _prefetch():
            pltpu.async_copy(table.at[src_v.at[nj]],
                             rows_v.at[lax.rem(nj, _NB)], gsem)
        return 0
    lax.fori_loop(0, _KC, body, 0)
    pltpu.make_async_copy(table.at[pl.ds(0, _CB)], rows_v.at[0], ssem).wait()
    _acc_epilogue(cid, sid, dump_v, acc_sh, out)


_sc_deg = pl.kernel(
    _sc_deg_body,
    out_type=jax.ShapeDtypeStruct((_NC, _NS, _RPT, 16), jnp.float32),
    mesh=_mesh,
    compiler_params=pltpu.CompilerParams(use_tc_tiling_on_sc=False),
    scratch_types=[
        pltpu.VMEM((_KC, _CB), jnp.int32),
        pltpu.VMEM((_CB, 16), jnp.float32),
        pltpu.VMEM((_RPT, 16), jnp.float32),
        pltpu.VMEM_SHARED((_ACCN, 16), jnp.float32),
        pltpu.SemaphoreType.DMA,
    ],
)

_sc_gs = pl.kernel(
    _sc_gs_body,
    out_type=jax.ShapeDtypeStruct((_NC, _NS, _RPT, 16), jnp.float32),
    mesh=_mesh,
    compiler_params=pltpu.CompilerParams(use_tc_tiling_on_sc=False),
    scratch_types=[
        pltpu.VMEM((_KC, _CB), jnp.int32),
        pltpu.VMEM((_KC, _CB), jnp.int32),
        pltpu.VMEM((_NB, _CB, 16), jnp.float32),
        pltpu.VMEM((_RPT, 16), jnp.float32),
        pltpu.VMEM_SHARED((_ACCN, 16), jnp.float32),
        pltpu.SemaphoreType.DMA,
        pltpu.SemaphoreType.DMA,
    ],
)


_RB = 1000
_GRID = _N // _RB


def _tc_prep_body(x_ref, w1_ref, degp_ref, dinv_ref, xws_ref):
    deg = degp_ref[0, :, 0:1] + degp_ref[1, :, 0:1] + 1.0
    dinv = lax.rsqrt(deg)
    xw = jnp.dot(x_ref[...], w1_ref[...], preferred_element_type=jnp.float32)
    dinv_ref[...] = dinv
    xws_ref[...] = xw * dinv


_tc_prep = pl.pallas_call(
    _tc_prep_body,
    grid=(_GRID,),
    in_specs=[
        pl.BlockSpec((_RB, _D), lambda i: (i, 0)),
        pl.BlockSpec((_D, _H), lambda i: (0, 0)),
        pl.BlockSpec((_NC, _RB, 16), lambda i: (0, i, 0)),
    ],
    out_specs=[
        pl.BlockSpec((_RB, 1), lambda i: (i, 0)),
        pl.BlockSpec((_RB, _H), lambda i: (i, 0)),
    ],
    out_shape=[
        jax.ShapeDtypeStruct((_N, 1), jnp.float32),
        jax.ShapeDtypeStruct((_N, _H), jnp.float32),
    ],
)


def _tc_layer_body(p_ref, dinv_ref, xws_ref, b1_ref, w2_ref, h_ref, hws_ref):
    dinv = dinv_ref[...]
    s = (p_ref[0] + p_ref[1] + xws_ref[...]) * dinv
    h = jnp.maximum(s + b1_ref[...], 0.0)
    h_ref[...] = h
    hws_ref[...] = jnp.dot(h, w2_ref[...],
                           preferred_element_type=jnp.float32) * dinv


_tc_layer = pl.pallas_call(
    _tc_layer_body,
    grid=(_GRID,),
    in_specs=[
        pl.BlockSpec((_NC, _RB, 16), lambda i: (0, i, 0)),
        pl.BlockSpec((_RB, 1), lambda i: (i, 0)),
        pl.BlockSpec((_RB, _H), lambda i: (i, 0)),
        pl.BlockSpec((1, _H), lambda i: (0, 0)),
        pl.BlockSpec((_H, 16), lambda i: (0, 0)),
    ],
    out_specs=[
        pl.BlockSpec((_RB, _H), lambda i: (i, 0)),
        pl.BlockSpec((_RB, 16), lambda i: (i, 0)),
    ],
    out_shape=[
        jax.ShapeDtypeStruct((_N, _H), jnp.float32),
        jax.ShapeDtypeStruct((_N, 16), jnp.float32),
    ],
)


def _tc_final_body(q_ref, dinv_ref, hws_ref, b2_ref, z_ref):
    z_ref[...] = ((q_ref[0] + q_ref[1] + hws_ref[...]) * dinv_ref[...]
                  + b2_ref[...])


_tc_final = pl.pallas_call(
    _tc_final_body,
    grid=(_GRID,),
    in_specs=[
        pl.BlockSpec((_NC, _RB, 16), lambda i: (0, i, 0)),
        pl.BlockSpec((_RB, 1), lambda i: (i, 0)),
        pl.BlockSpec((_RB, 16), lambda i: (i, 0)),
        pl.BlockSpec((1, 16), lambda i: (0, 0)),
    ],
    out_specs=pl.BlockSpec((_RB, 16), lambda i: (i, 0)),
    out_shape=jax.ShapeDtypeStruct((_N, 16), jnp.float32),
)


def kernel(x, edge_index, W1, b1, W2, b2):
    ei = edge_index.astype(jnp.int32)
    npad = _EPAD - _E
    pad_src = jnp.arange(npad, dtype=jnp.int32) % _N
    src3 = jnp.concatenate([ei[0], pad_src]).reshape(_NW, _KC, _CB)
    pad_dst = _N + (jnp.arange(npad, dtype=jnp.int32) % _NTRASH)
    dst3 = jnp.concatenate([ei[1], pad_dst]).reshape(_NW, _KC, _CB)

    b1r = b1.reshape(1, _H)
    W2p = jnp.zeros((_H, 16), jnp.float32).at[:, :_C].set(W2)
    b2p = jnp.zeros((1, 16), jnp.float32).at[0, :_C].set(b2)

    degp = _sc_deg(dst3).reshape(_NC, _N, 16)
    dinv, xws = _tc_prep(x, W1, degp)
    p = _sc_gs(xws, src3, dst3).reshape(_NC, _N, 16)
    h, hws = _tc_layer(p, dinv, xws, b1r, W2p)
    q = _sc_gs(hws, src3, dst3).reshape(_NC, _N, 16)
    z = _tc_final(q, dinv, hws, b2p)[:, :_C]
    return (h, z)


# speedup vs baseline: 1.5172x; 1.1676x over previous
import functools

import jax
import jax.numpy as jnp
from jax import lax
from jax.experimental import pallas as pl
from jax.experimental.pallas import tpu as pltpu
from jax.experimental.pallas import tpu_sc as plsc

_N = 10000
_E = 320000
_D = 128
_H = 16
_C = 7

_NC = 2
_NS = 16
_NW = _NC * _NS
_CB = 128
_KC = 80
_EPT = _KC * _CB
_EPAD = _NW * _EPT
_NTRASH = 512
_RPT = _N // _NS

_SLAG = 4
_ACCN = _N + _NTRASH

_mesh = plsc.VectorSubcoreMesh(core_axis_name="c", subcore_axis_name="s")


def _zero_rows(buf, nrows):
    def body(i, _):
        buf[i] = jnp.zeros((16,), jnp.float32)
        return 0
    lax.fori_loop(0, nrows, body, 0)


def _acc_prologue(sid, dump_v, acc_sh):
    _zero_rows(dump_v, _RPT)
    pltpu.sync_copy(dump_v, acc_sh.at[pl.ds(sid * _RPT, _RPT)])
    plsc.subcore_barrier()


def _acc_epilogue(cid, sid, dump_v, acc_sh, out):
    plsc.subcore_barrier()
    pltpu.sync_copy(acc_sh.at[pl.ds(sid * _RPT, _RPT)], dump_v)
    pltpu.sync_copy(dump_v, out.at[cid, sid])


def _sc_deg_body(dst_h, out, dst_v, rows_v, dump_v, acc_sh, ssem):
    cid = lax.axis_index("c")
    sid = lax.axis_index("s")
    wid = sid * _NC + cid
    _acc_prologue(sid, dump_v, acc_sh)
    pltpu.sync_copy(dst_h.at[wid], dst_v)

    def fill(i, _):
        rows_v[i] = jnp.ones((16,), jnp.float32)
        return 0
    lax.fori_loop(0, _CB, fill, 0)

    def body(j, _):
        pltpu.async_copy(rows_v, acc_sh.at[dst_v.at[j]], ssem, add=True)

        @pl.when(j >= _SLAG)
        def _wait_old():
            pltpu.make_async_copy(out.at[0, 0, pl.ds(0, _CB)], rows_v,
                                  ssem).wait()
        return 0
    lax.fori_loop(0, _KC, body, 0)

    def drain(j, _):
        pltpu.make_async_copy(out.at[0, 0, pl.ds(0, _CB)], rows_v,
                              ssem).wait()
        return 0
    lax.fori_loop(0, _SLAG, drain, 0)
    _acc_epilogue(cid, sid, dump_v, acc_sh, out)


_NB = 8


def _sc_gs_body(table, src_h, dst_h, out, src_v, dst_v, rows_v, dump_v,
                acc_sh, gsem, ssem):
    cid = lax.axis_index("c")
    sid = lax.axis_index("s")
    wid = sid * _NC + cid
    _acc_prologue(sid, dump_v, acc_sh)
    pltpu.sync_copy(src_h.at[wid], src_v)
    pltpu.sync_copy(dst_h.at[wid], dst_v)

    for b in range(_NB - 1):
        pltpu.async_copy(table.at[src_v.at[b]], rows_v.at[b], gsem)

    def body(j, _):
        b = lax.rem(j, _NB)
        pltpu.make_async_copy(table.at[pl.ds(0, _CB)], rows_v.at[b],
                              gsem).wait()
        pltpu.async_copy(rows_v.at[b], acc_sh.at[dst_v.at[j]], ssem,
                         add=True)

        @pl.when(j >= 1)
        def _wait_prev_scatter():
            pltpu.make_async_copy(table.at[pl.ds(0, _CB)], rows_v.at[0],
                                  ssem).wait()

        nj = j + _NB - 1

        @pl.when(nj < _KC)
        def _prefetch():
            pltpu.async_copy(table.at[src_v.at[nj]],
                             rows_v.at[lax.rem(nj, _NB)], gsem)
        return 0
    lax.fori_loop(0, _KC, body, 0)
    pltpu.make_async_copy(table.at[pl.ds(0, _CB)], rows_v.at[0], ssem).wait()
    _acc_epilogue(cid, sid, dump_v, acc_sh, out)


_sc_deg = pl.kernel(
    _sc_deg_body,
    out_type=jax.ShapeDtypeStruct((_NC, _NS, _RPT, 16), jnp.float32),
    mesh=_mesh,
    compiler_params=pltpu.CompilerParams(use_tc_tiling_on_sc=False),
    scratch_types=[
        pltpu.VMEM((_KC, _CB), jnp.int32),
        pltpu.VMEM((_CB, 16), jnp.float32),
        pltpu.VMEM((_RPT, 16), jnp.float32),
        pltpu.VMEM_SHARED((_ACCN, 16), jnp.float32),
        pltpu.SemaphoreType.DMA,
    ],
)

_sc_gs = pl.kernel(
    _sc_gs_body,
    out_type=jax.ShapeDtypeStruct((_NC, _NS, _RPT, 16), jnp.float32),
    mesh=_mesh,
    compiler_params=pltpu.CompilerParams(use_tc_tiling_on_sc=False),
    scratch_types=[
        pltpu.VMEM((_KC, _CB), jnp.int32),
        pltpu.VMEM((_KC, _CB), jnp.int32),
        pltpu.VMEM((_NB, _CB, 16), jnp.float32),
        pltpu.VMEM((_RPT, 16), jnp.float32),
        pltpu.VMEM_SHARED((_ACCN, 16), jnp.float32),
        pltpu.SemaphoreType.DMA,
        pltpu.SemaphoreType.DMA,
    ],
)


_RB = 1000
_GRID = _N // _RB


def _tc_prep_body(x_ref, w1_ref, degp_ref, dinv_ref, xws_ref):
    deg = degp_ref[0, :, 0:1] + degp_ref[1, :, 0:1] + 1.0
    dinv = lax.rsqrt(deg)
    xw = jnp.dot(x_ref[...], w1_ref[...], preferred_element_type=jnp.float32)
    dinv_ref[...] = dinv
    xws_ref[...] = xw * dinv


_tc_prep = pl.pallas_call(
    _tc_prep_body,
    grid=(_GRID,),
    in_specs=[
        pl.BlockSpec((_RB, _D), lambda i: (i, 0)),
        pl.BlockSpec((_D, _H), lambda i: (0, 0)),
        pl.BlockSpec((_NC, _RB, 16), lambda i: (0, i, 0)),
    ],
    out_specs=[
        pl.BlockSpec((_RB, 1), lambda i: (i, 0)),
        pl.BlockSpec((_RB, _H), lambda i: (i, 0)),
    ],
    out_shape=[
        jax.ShapeDtypeStruct((_N, 1), jnp.float32),
        jax.ShapeDtypeStruct((_N, _H), jnp.float32),
    ],
)


def _tc_layer_body(p_ref, dinv_ref, xws_ref, b1_ref, w2_ref, h_ref, hws_ref):
    dinv = dinv_ref[...]
    s = (p_ref[0] + p_ref[1] + xws_ref[...]) * dinv
    h = jnp.maximum(s + b1_ref[...], 0.0)
    h_ref[...] = h
    hws_ref[...] = jnp.dot(h, w2_ref[...],
                           preferred_element_type=jnp.float32) * dinv


_tc_layer = pl.pallas_call(
    _tc_layer_body,
    grid=(_GRID,),
    in_specs=[
        pl.BlockSpec((_NC, _RB, 16), lambda i: (0, i, 0)),
        pl.BlockSpec((_RB, 1), lambda i: (i, 0)),
        pl.BlockSpec((_RB, _H), lambda i: (i, 0)),
        pl.BlockSpec((1, _H), lambda i: (0, 0)),
        pl.BlockSpec((_H, 16), lambda i: (0, 0)),
    ],
    out_specs=[
        pl.BlockSpec((_RB, _H), lambda i: (i, 0)),
        pl.BlockSpec((_RB, 16), lambda i: (i, 0)),
    ],
    out_shape=[
        jax.ShapeDtypeStruct((_N, _H), jnp.float32),
        jax.ShapeDtypeStruct((_N, 16), jnp.float32),
    ],
)


def _tc_final_body(q_ref, dinv_ref, hws_ref, b2_ref, z_ref):
    z_ref[...] = ((q_ref[0] + q_ref[1] + hws_ref[...]) * dinv_ref[...]
                  + b2_ref[...])


_tc_final = pl.pallas_call(
    _tc_final_body,
    grid=(_GRID,),
    in_specs=[
        pl.BlockSpec((_NC, _RB, 16), lambda i: (0, i, 0)),
        pl.BlockSpec((_RB, 1), lambda i: (i, 0)),
        pl.BlockSpec((_RB, 16), lambda i: (i, 0)),
        pl.BlockSpec((1, 16), lambda i: (0, 0)),
    ],
    out_specs=pl.BlockSpec((_RB, 16), lambda i: (i, 0)),
    out_shape=jax.ShapeDtypeStruct((_N, 16), jnp.float32),
)


def kernel(x, edge_index, W1, b1, W2, b2):
    ei = edge_index.astype(jnp.int32)
    npad = _EPAD - _E
    pad_src = jnp.arange(npad, dtype=jnp.int32) % _N
    src3 = jnp.concatenate([ei[0], pad_src]).reshape(_NW, _KC, _CB)
    pad_dst = _N + (jnp.arange(npad, dtype=jnp.int32) % _NTRASH)
    dst3 = jnp.concatenate([ei[1], pad_dst]).reshape(_NW, _KC, _CB)

    b1r = b1.reshape(1, _H)
    W2p = jnp.zeros((_H, 16), jnp.float32).at[:, :_C].set(W2)
    b2p = jnp.zeros((1, 16), jnp.float32).at[0, :_C].set(b2)

    degp = _sc_deg(dst3).reshape(_NC, _N, 16)
    dinv, xws = _tc_prep(x, W1, degp)
    p = _sc_gs(xws, src3, dst3).reshape(_NC, _N, 16)
    h, hws = _tc_layer(p, dinv, xws, b1r, W2p)
    q = _sc_gs(hws, src3, dst3).reshape(_NC, _N, 16)
    z = _tc_final(q, dinv, hws, b2p)[:, :_C]
    return (h, z)


# speedup vs baseline: 1.5977x; 1.0531x over previous
import functools

import jax
import jax.numpy as jnp
from jax import lax
from jax.experimental import pallas as pl
from jax.experimental.pallas import tpu as pltpu
from jax.experimental.pallas import tpu_sc as plsc

_N = 10000
_E = 320000
_D = 128
_H = 16
_C = 7

_NC = 2
_NS = 16
_NW = _NC * _NS
_CB = 128
_KC = 80
_EPT = _KC * _CB
_EPAD = _NW * _EPT
_NTRASH = 512
_RPT = _N // _NS

_SLAG = 4
_ACCN = _N + _NTRASH

_mesh = plsc.VectorSubcoreMesh(core_axis_name="c", subcore_axis_name="s")


def _zero_rows(buf, nrows):
    def body(i, _):
        buf[i] = jnp.zeros((16,), jnp.float32)
        return 0
    lax.fori_loop(0, nrows, body, 0)


def _acc_prologue(sid, dump_v, acc_sh):
    _zero_rows(dump_v, _RPT)
    pltpu.sync_copy(dump_v, acc_sh.at[pl.ds(sid * _RPT, _RPT)])
    plsc.subcore_barrier()


def _acc_epilogue(cid, sid, dump_v, acc_sh, out):
    plsc.subcore_barrier()
    pltpu.sync_copy(acc_sh.at[pl.ds(sid * _RPT, _RPT)], dump_v)
    pltpu.sync_copy(dump_v, out.at[cid, sid])


def _sc_deg_body(dst_h, out, dst_v, rows_v, dump_v, acc_sh, ssem):
    cid = lax.axis_index("c")
    sid = lax.axis_index("s")
    wid = sid * _NC + cid
    _acc_prologue(sid, dump_v, acc_sh)
    pltpu.sync_copy(dst_h.at[wid], dst_v)

    def fill(i, _):
        rows_v[i] = jnp.ones((16,), jnp.float32)
        return 0
    lax.fori_loop(0, _CB, fill, 0)

    def body(j, _):
        pltpu.async_copy(rows_v, acc_sh.at[dst_v.at[j]], ssem, add=True)

        @pl.when(j >= _SLAG)
        def _wait_old():
            pltpu.make_async_copy(out.at[0, 0, pl.ds(0, _CB)], rows_v,
                                  ssem).wait()
        return 0
    lax.fori_loop(0, _KC, body, 0)

    def drain(j, _):
        pltpu.make_async_copy(out.at[0, 0, pl.ds(0, _CB)], rows_v,
                              ssem).wait()
        return 0
    lax.fori_loop(0, _SLAG, drain, 0)
    _acc_epilogue(cid, sid, dump_v, acc_sh, out)


_NB = 16


def _sc_gs_body(table, src_h, dst_h, out, src_v, dst_v, rows_v, dump_v,
                acc_sh, gsem, ssem):
    cid = lax.axis_index("c")
    sid = lax.axis_index("s")
    wid = sid * _NC + cid
    _acc_prologue(sid, dump_v, acc_sh)
    pltpu.sync_copy(src_h.at[wid], src_v)
    pltpu.sync_copy(dst_h.at[wid], dst_v)

    for b in range(_NB - 1):
        pltpu.async_copy(table.at[src_v.at[b]], rows_v.at[b], gsem)

    def body(j, _):
        b = lax.rem(j, _NB)
        pltpu.make_async_copy(table.at[pl.ds(0, _CB)], rows_v.at[b],
                              gsem).wait()
        pltpu.async_copy(rows_v.at[b], acc_sh.at[dst_v.at[j]], ssem,
                         add=True)

        @pl.when(j >= 1)
        def _wait_prev_scatter():
            pltpu.make_async_copy(table.at[pl.ds(0, _CB)], rows_v.at[0],
                                  ssem).wait()

        nj = j + _NB - 1

        @pl.when(nj < _KC)
        def _prefetch():
            pltpu.async_copy(table.at[src_v.at[nj]],
                             rows_v.at[lax.rem(nj, _NB)], gsem)
        return 0
    lax.fori_loop(0, _KC, body, 0)
    pltpu.make_async_copy(table.at[pl.ds(0, _CB)], rows_v.at[0], ssem).wait()
    _acc_epilogue(cid, sid, dump_v, acc_sh, out)


_sc_deg = pl.kernel(
    _sc_deg_body,
    out_type=jax.ShapeDtypeStruct((_NC, _NS, _RPT, 16), jnp.float32),
    mesh=_mesh,
    compiler_params=pltpu.CompilerParams(use_tc_tiling_on_sc=False),
    scratch_types=[
        pltpu.VMEM((_KC, _CB), jnp.int32),
        pltpu.VMEM((_CB, 16), jnp.float32),
        pltpu.VMEM((_RPT, 16), jnp.float32),
        pltpu.VMEM_SHARED((_ACCN, 16), jnp.float32),
        pltpu.SemaphoreType.DMA,
    ],
)

_sc_gs = pl.kernel(
    _sc_gs_body,
    out_type=jax.ShapeDtypeStruct((_NC, _NS, _RPT, 16), jnp.float32),
    mesh=_mesh,
    compiler_params=pltpu.CompilerParams(use_tc_tiling_on_sc=False),
    scratch_types=[
        pltpu.VMEM((_KC, _CB), jnp.int32),
        pltpu.VMEM((_KC, _CB), jnp.int32),
        pltpu.VMEM((_NB, _CB, 16), jnp.float32),
        pltpu.VMEM((_RPT, 16), jnp.float32),
        pltpu.VMEM_SHARED((_ACCN, 16), jnp.float32),
        pltpu.SemaphoreType.DMA,
        pltpu.SemaphoreType.DMA,
    ],
)


_RB = 1000
_GRID = _N // _RB


def _tc_prep_body(x_ref, w1_ref, degp_ref, dinv_ref, xws_ref):
    deg = degp_ref[0, :, 0:1] + degp_ref[1, :, 0:1] + 1.0
    dinv = lax.rsqrt(deg)
    xw = jnp.dot(x_ref[...], w1_ref[...], preferred_element_type=jnp.float32)
    dinv_ref[...] = dinv
    xws_ref[...] = xw * dinv


_tc_prep = pl.pallas_call(
    _tc_prep_body,
    grid=(_GRID,),
    in_specs=[
        pl.BlockSpec((_RB, _D), lambda i: (i, 0)),
        pl.BlockSpec((_D, _H), lambda i: (0, 0)),
        pl.BlockSpec((_NC, _RB, 16), lambda i: (0, i, 0)),
    ],
    out_specs=[
        pl.BlockSpec((_RB, 1), lambda i: (i, 0)),
        pl.BlockSpec((_RB, _H), lambda i: (i, 0)),
    ],
    out_shape=[
        jax.ShapeDtypeStruct((_N, 1), jnp.float32),
        jax.ShapeDtypeStruct((_N, _H), jnp.float32),
    ],
)


def _tc_layer_body(p_ref, dinv_ref, xws_ref, b1_ref, w2_ref, h_ref, hws_ref):
    dinv = dinv_ref[...]
    s = (p_ref[0] + p_ref[1] + xws_ref[...]) * dinv
    h = jnp.maximum(s + b1_ref[...], 0.0)
    h_ref[...] = h
    hws_ref[...] = jnp.dot(h, w2_ref[...],
                           preferred_element_type=jnp.float32) * dinv


_tc_layer = pl.pallas_call(
    _tc_layer_body,
    grid=(_GRID,),
    in_specs=[
        pl.BlockSpec((_NC, _RB, 16), lambda i: (0, i, 0)),
        pl.BlockSpec((_RB, 1), lambda i: (i, 0)),
        pl.BlockSpec((_RB, _H), lambda i: (i, 0)),
        pl.BlockSpec((1, _H), lambda i: (0, 0)),
        pl.BlockSpec((_H, 16), lambda i: (0, 0)),
    ],
    out_specs=[
        pl.BlockSpec((_RB, _H), lambda i: (i, 0)),
        pl.BlockSpec((_RB, 16), lambda i: (i, 0)),
    ],
    out_shape=[
        jax.ShapeDtypeStruct((_N, _H), jnp.float32),
        jax.ShapeDtypeStruct((_N, 16), jnp.float32),
    ],
)


def _tc_final_body(q_ref, dinv_ref, hws_ref, b2_ref, z_ref):
    z_ref[...] = ((q_ref[0] + q_ref[1] + hws_ref[...]) * dinv_ref[...]
                  + b2_ref[...])


_tc_final = pl.pallas_call(
    _tc_final_body,
    grid=(_GRID,),
    in_specs=[
        pl.BlockSpec((_NC, _RB, 16), lambda i: (0, i, 0)),
        pl.BlockSpec((_RB, 1), lambda i: (i, 0)),
        pl.BlockSpec((_RB, 16), lambda i: (i, 0)),
        pl.BlockSpec((1, 16), lambda i: (0, 0)),
    ],
    out_specs=pl.BlockSpec((_RB, 16), lambda i: (i, 0)),
    out_shape=jax.ShapeDtypeStruct((_N, 16), jnp.float32),
)


def kernel(x, edge_index, W1, b1, W2, b2):
    ei = edge_index.astype(jnp.int32)
    npad = _EPAD - _E
    pad_src = jnp.arange(npad, dtype=jnp.int32) % _N
    src3 = jnp.concatenate([ei[0], pad_src]).reshape(_NW, _KC, _CB)
    pad_dst = _N + (jnp.arange(npad, dtype=jnp.int32) % _NTRASH)
    dst3 = jnp.concatenate([ei[1], pad_dst]).reshape(_NW, _KC, _CB)

    b1r = b1.reshape(1, _H)
    W2p = jnp.zeros((_H, 16), jnp.float32).at[:, :_C].set(W2)
    b2p = jnp.zeros((1, 16), jnp.float32).at[0, :_C].set(b2)

    degp = _sc_deg(dst3).reshape(_NC, _N, 16)
    dinv, xws = _tc_prep(x, W1, degp)
    p = _sc_gs(xws, src3, dst3).reshape(_NC, _N, 16)
    h, hws = _tc_layer(p, dinv, xws, b1r, W2p)
    q = _sc_gs(hws, src3, dst3).reshape(_NC, _N, 16)
    z = _tc_final(q, dinv, hws, b2p)[:, :_C]
    return (h, z)
